# baseline (speedup 1.0000x reference)
import jax
import jax.numpy as jnp
from jax.experimental import pallas as pl
from jax.experimental.pallas import tpu as pltpu

_N_ACT = 6
_BB = 32


def _dqn_kernel(xs_ref, w1_ref, b1_ref, w2_ref, b2_ref, w3_ref, b3_ref,
                wl1_ref, bl1_ref, wl2_ref, bl2_ref, o_ref):
    bb = xs_ref.shape[0]
    x = xs_ref[...].reshape(bb, 21, 4, 11, 32)

    def colsel(v, q2):
        if q2 == 0:
            return v[:, :, 0:10, :]
        return jnp.concatenate([v[:, :, 0:10, 16:], v[:, :, 1:11, :16]],
                               axis=-1)

    m = [[None, None], [None, None]]
    for q2 in (0, 1):
        p = jnp.concatenate(
            [colsel(x[:, bi:bi + 20, r], q2) for bi in (0, 1) for r in range(4)],
            axis=-1)
        a = jnp.dot(p.reshape(bb * 200, 256), w1_ref[...],
                    preferred_element_type=jnp.float32)
        a = jnp.maximum(a + b1_ref[...], 0.0).astype(jnp.bfloat16)
        a = a.reshape(bb, 20, 10, 32).reshape(bb, 10, 2, 10, 32)
        m[0][q2] = a[:, :, 0]
        m[1][q2] = a[:, :, 1]

    p = jnp.concatenate(
        [m[i % 2][j % 2][:, i // 2:i // 2 + 9, j // 2:j // 2 + 9, :]
         for i in range(4) for j in range(4)], axis=-1)
    a = jnp.dot(p.reshape(bb * 81, 512), w2_ref[...],
                preferred_element_type=jnp.float32)
    a = jnp.maximum(a + b2_ref[...], 0.0).astype(jnp.bfloat16)
    a = a.reshape(bb, 9, 9, 64)

    p = jnp.concatenate(
        [a[:, i:i + 7, j:j + 7, :] for i in range(3) for j in range(3)],
        axis=-1)
    a = jnp.dot(p.reshape(bb * 49, 576), w3_ref[...],
                preferred_element_type=jnp.float32)
    a = jnp.maximum(a + b3_ref[...], 0.0).astype(jnp.bfloat16)

    a = a.reshape(bb, 49, 64)
    flat = jnp.concatenate([a[:, p_, :] for p_ in range(49)], axis=-1)

    h = jnp.dot(flat, wl1_ref[...], preferred_element_type=jnp.float32)
    h = jnp.maximum(h + bl1_ref[...], 0.0).astype(jnp.bfloat16)
    q = jnp.dot(h, wl2_ref[...], preferred_element_type=jnp.float32)
    o_ref[...] = q + bl2_ref[...]


def kernel(w1, b1, w2, b2, w3, b3, wl1, bl1, wl2, bl2, x):
    B = x.shape[0]
    Bp = (B + _BB - 1) // _BB * _BB
    if Bp != B:
        x = jnp.pad(x, ((0, Bp - B), (0, 0), (0, 0), (0, 0)))

    xn = jnp.transpose(x, (0, 2, 3, 1)).astype(jnp.bfloat16)
    xn = jnp.pad(xn, ((0, 0), (0, 0), (0, 4), (0, 0)))
    xs = xn.reshape(Bp, 84, 11, 32)

    q = pl.pallas_call(
        _dqn_kernel,
        out_shape=jax.ShapeDtypeStruct((Bp, 128), jnp.float32),
        grid=(Bp // _BB,),
        in_specs=[
            pl.BlockSpec((_BB, 84, 11, 32), lambda i: (i, 0, 0, 0)),
            pl.BlockSpec((256, 32), lambda i: (0, 0)),
            pl.BlockSpec((1, 32), lambda i: (0, 0)),
            pl.BlockSpec((512, 64), lambda i: (0, 0)),
            pl.BlockSpec((1, 64), lambda i: (0, 0)),
            pl.BlockSpec((576, 64), lambda i: (0, 0)),
            pl.BlockSpec((1, 64), lambda i: (0, 0)),
            pl.BlockSpec((3136, 512), lambda i: (0, 0)),
            pl.BlockSpec((1, 512), lambda i: (0, 0)),
            pl.BlockSpec((512, 128), lambda i: (0, 0)),
            pl.BlockSpec((1, 128), lambda i: (0, 0)),
        ],
        out_specs=pl.BlockSpec((_BB, 128), lambda i: (i, 0)),
        compiler_params=pltpu.CompilerParams(
            dimension_semantics=("parallel",),
            vmem_limit_bytes=64 * 1024 * 1024,
        ),
    )(xs, w1, b1.reshape(1, 32), w2, b2.reshape(1, 64), w3, b3.reshape(1, 64),
      wl1, bl1.reshape(1, 512), wl2, bl2.reshape(1, 128))
    return q[:B, :_N_ACT]


# speedup vs baseline: 1.6130x; 1.6130x over previous
import numpy as np

import jax
import jax.numpy as jnp
from jax.experimental import pallas as pl
from jax.experimental.pallas import tpu as pltpu

_N_ACT = 6
_BB = 32


def _w1_perm():
    perm = np.empty(256, np.int32)
    for bi in range(2):
        for j in range(8):
            for r in range(4):
                for c in range(4):
                    i = 4 * bi + r
                    perm[bi * 128 + j * 16 + r * 4 + c] = (i * 8 + j) * 4 + c
    return perm


_PERM1 = _w1_perm()


def _dqn_kernel(xs_ref, w1_ref, b1_ref, w2_ref, b2_ref, w3_ref, b3_ref,
                wl1_ref, bl1_ref, wl2_ref, bl2_ref, o_ref):
    bb = xs_ref.shape[0]
    x = xs_ref[...].reshape(bb, 11, 2, 11, 128)

    def rowsel(off):
        if off < 2:
            return x[:, 0:10, off]
        return x[:, 1:11, 0]

    def colsel(xr, q2):
        if q2 == 0:
            return xr[:, :, 0:10, :]
        return jnp.concatenate([xr[:, :, 0:10, 64:], xr[:, :, 1:11, :64]],
                               axis=-1)

    m = [[None, None], [None, None]]
    for r2 in (0, 1):
        for q2 in (0, 1):
            p = jnp.concatenate(
                [colsel(rowsel(r2 + bi), q2) for bi in (0, 1)], axis=-1)
            a = jnp.dot(p.reshape(bb * 100, 256), w1_ref[...],
                        preferred_element_type=jnp.float32)
            a = jnp.maximum(a + b1_ref[...], 0.0).astype(jnp.bfloat16)
            m[r2][q2] = a.reshape(bb, 10, 10, 32)

    p = jnp.concatenate(
        [m[i % 2][j % 2][:, i // 2:i // 2 + 9, j // 2:j // 2 + 9, :]
         for i in range(4) for j in range(4)], axis=-1)
    a = jnp.dot(p.reshape(bb * 81, 512), w2_ref[...],
                preferred_element_type=jnp.float32)
    a = jnp.maximum(a + b2_ref[...], 0.0).astype(jnp.bfloat16)
    a = a.reshape(bb, 9, 9, 64)

    p = jnp.concatenate(
        [a[:, i:i + 7, j:j + 7, :] for i in range(3) for j in range(3)],
        axis=-1)
    a = jnp.dot(p.reshape(bb * 49, 576), w3_ref[...],
                preferred_element_type=jnp.float32)
    a = jnp.maximum(a + b3_ref[...], 0.0).astype(jnp.bfloat16)

    a = a.reshape(bb, 49, 64)
    flat = jnp.concatenate([a[:, p, :] for p in range(49)], axis=-1)

    h = jnp.dot(flat, wl1_ref[...], preferred_element_type=jnp.float32)
    h = jnp.maximum(h + bl1_ref[...], 0.0).astype(jnp.bfloat16)
    q = jnp.dot(h, wl2_ref[...], preferred_element_type=jnp.float32)
    o_ref[...] = q + bl2_ref[...]


def kernel(w1, b1, w2, b2, w3, b3, wl1, bl1, wl2, bl2, x):
    B = x.shape[0]
    Bp = (B + _BB - 1) // _BB * _BB
    if Bp != B:
        x = jnp.pad(x, ((0, Bp - B), (0, 0), (0, 0), (0, 0)))

    xp = jnp.pad(x.astype(jnp.bfloat16), ((0, 0), (0, 0), (0, 4), (0, 4)))
    xs = jnp.transpose(xp.reshape(Bp, 4, 22, 4, 11, 8),
                       (0, 2, 4, 5, 3, 1)).reshape(Bp, 22, 11, 128)
    w1p = w1[_PERM1]

    q = pl.pallas_call(
        _dqn_kernel,
        out_shape=jax.ShapeDtypeStruct((Bp, 128), jnp.float32),
        grid=(Bp // _BB,),
        in_specs=[
            pl.BlockSpec((_BB, 22, 11, 128), lambda i: (i, 0, 0, 0)),
            pl.BlockSpec((256, 32), lambda i: (0, 0)),
            pl.BlockSpec((1, 32), lambda i: (0, 0)),
            pl.BlockSpec((512, 64), lambda i: (0, 0)),
            pl.BlockSpec((1, 64), lambda i: (0, 0)),
            pl.BlockSpec((576, 64), lambda i: (0, 0)),
            pl.BlockSpec((1, 64), lambda i: (0, 0)),
            pl.BlockSpec((3136, 512), lambda i: (0, 0)),
            pl.BlockSpec((1, 512), lambda i: (0, 0)),
            pl.BlockSpec((512, 128), lambda i: (0, 0)),
            pl.BlockSpec((1, 128), lambda i: (0, 0)),
        ],
        out_specs=pl.BlockSpec((_BB, 128), lambda i: (i, 0)),
        compiler_params=pltpu.CompilerParams(
            dimension_semantics=("parallel",),
            vmem_limit_bytes=64 * 1024 * 1024,
        ),
    )(xs, w1p, b1.reshape(1, 32), w2, b2.reshape(1, 64), w3, b3.reshape(1, 64),
      wl1, bl1.reshape(1, 512), wl2, bl2.reshape(1, 128))
    return q[:B, :_N_ACT]
